# baseline (speedup 1.0000x reference)
import functools

import jax
import jax.numpy as jnp
from jax import lax
from jax.experimental import pallas as pl
from jax.experimental.pallas import tpu as pltpu
from jax.experimental.pallas import tpu_sc as plsc

K = 50
N = 8192
NW = 32
RPW = N // NW
PADK = 128
NEG = float("-inf")
BIG = 1 << 30
RB = 256
CAPA = 544
SEG = 256
BOOT = 64
ROWPAD = N + SEG

BT = 512


def _ssum_body(a_ref, b_ref, o_ref):
    o_ref[...] = a_ref[...] + b_ref[...].T


def _ssum(m):
    g = N // BT
    return pl.pallas_call(
        _ssum_body,
        grid=(g, g),
        in_specs=[
            pl.BlockSpec((BT, BT), lambda i, j: (i, j)),
            pl.BlockSpec((BT, BT), lambda i, j: (j, i)),
        ],
        out_specs=pl.BlockSpec((BT, BT), lambda i, j: (i, j)),
        out_shape=jax.ShapeDtypeStruct((N, N), jnp.float32),
    )(m, m)


def _iota16():
    return lax.iota(jnp.int32, 16)


def _topk_sc(ssum):
    mesh = plsc.VectorSubcoreMesh(core_axis_name="c", subcore_axis_name="s")

    @functools.partial(
        pl.kernel,
        mesh=mesh,
        out_type=[
            jax.ShapeDtypeStruct((N, PADK), jnp.float32),
            jax.ShapeDtypeStruct((N, PADK), jnp.int32),
        ],
        scratch_types=[
            pltpu.VMEM((ROWPAD,), jnp.float32),
            pltpu.VMEM((ROWPAD,), jnp.float32),
            pltpu.VMEM((CAPA,), jnp.float32),
            pltpu.VMEM((CAPA,), jnp.int32),
            pltpu.VMEM((PADK,), jnp.float32),
            pltpu.VMEM((PADK,), jnp.int32),
            pltpu.SemaphoreType.DMA,
            pltpu.SemaphoreType.DMA,
        ],
        compiler_params=pltpu.CompilerParams(needs_layout_passes=False),
    )
    def k(s_hbm, ov_hbm, oi_hbm, row0, row1, bufv, bufi, outv, outi,
          sem0, sem1):
        wid = lax.axis_index("s") * 2 + lax.axis_index("c")
        lane0 = _iota16() == 0

        def extract(ptr):
            bufv[pl.ds(ptr, 16)] = jnp.full((16,), NEG, jnp.float32)
            ngrp = ptr // 16 + 1

            def one(kk, _):
                def scan(g, c):
                    bv, bp = c
                    x = bufv[pl.ds(g * 16, 16)]
                    ci = bufi[pl.ds(g * 16, 16)]
                    sl = g * 16 + _iota16()
                    m = x > bv
                    return (jnp.where(m, x, bv),
                            jnp.where(m, sl * 16384 + ci, bp))

                bv, bp = lax.fori_loop(
                    0, ngrp, scan,
                    (jnp.full((16,), NEG, jnp.float32),
                     jnp.full((16,), BIG, jnp.int32)))
                mx = jnp.max(bv)
                tie = bv == mx
                pmin = jnp.min(jnp.where(tie, bp, BIG))
                smin = jnp.minimum(pmin >> 14, jnp.int32(CAPA - 17))
                col = pmin & 16383
                plsc.store_scatter(bufv, [jnp.full((16,), smin, jnp.int32)],
                                   jnp.full((16,), NEG, jnp.float32), mask=lane0)
                kk16 = jnp.full((16,), kk, jnp.int32)
                plsc.store_scatter(outv, [kk16],
                                   jnp.full((16,), mx, jnp.float32), mask=lane0)
                plsc.store_scatter(outi, [kk16],
                                   jnp.full((16,), col, jnp.int32), mask=lane0)
                return mx

            return lax.fori_loop(0, K, one, jnp.float32(NEG))

        def rebuild(ptr):
            thr = extract(ptr)
            for g in range(4):
                bufv[pl.ds(g * 16, 16)] = outv[pl.ds(g * 16, 16)]
                bufi[pl.ds(g * 16, 16)] = outi[pl.ds(g * 16, 16)]
            return thr, jnp.int32(K)

        def append(x, cols, mask, ptr):
            cnt = mask.astype(jnp.int32)
            pos = ptr + plsc.cumsum(cnt) - 1
            plsc.store_scatter(bufv, [pos], x, mask=mask)
            plsc.store_scatter(bufi, [pos], cols, mask=mask)
            return ptr + jnp.sum(cnt)

        def process(i, row):
            for t in range(16):
                row[pl.ds(i + t * 16, 16)] = jnp.full((16,), NEG,
                                                      jnp.float32)
            thr = jnp.float32(NEG)
            ptr = jnp.int32(0)
            for g in range(BOOT // 16):
                x = row[pl.ds(g * 16, 16)]
                ptr = append(x, g * 16 + _iota16(), x > thr, ptr)
            thr, ptr = rebuild(ptr)

            nseg = jnp.maximum(0, (i - BOOT + SEG - 1) // SEG)

            def seg_body(s, c):
                thr, ptr = c
                base = BOOT + s * SEG
                for g in range(SEG // 16):
                    x = row[pl.ds(base + g * 16, 16)]
                    mask = x > thr
                    ptr = lax.cond(
                        jnp.any(mask),
                        lambda xx=x, m=mask, b=base + g * 16, p=ptr:
                            append(xx, b + _iota16(), m, p),
                        lambda p=ptr: p)
                return lax.cond(ptr >= RB, lambda: rebuild(ptr),
                                lambda: (thr, ptr))

            thr, ptr = lax.fori_loop(0, nseg, seg_body, (thr, ptr))
            extract(ptr)

            for g in range(PADK // 16):
                pv = g * 16 + _iota16()
                pm = pv >= i
                outv[pl.ds(g * 16, 16)] = jnp.where(
                    pm, jnp.full((16,), NEG, jnp.float32),
                    outv[pl.ds(g * 16, 16)])
                outi[pl.ds(g * 16, 16)] = jnp.where(
                    pm, pv, outi[pl.ds(g * 16, 16)])
            pltpu.sync_copy(outv, ov_hbm.at[i])
            pltpu.sync_copy(outi, oi_hbm.at[i])

        pltpu.async_copy(s_hbm.at[wid], row0.at[pl.ds(0, N)], sem0)

        def pair(rp, carry):
            i0 = wid + NW * 2 * rp
            i1 = i0 + NW
            pltpu.async_copy(s_hbm.at[i1], row1.at[pl.ds(0, N)], sem1)
            pltpu.make_async_copy(s_hbm.at[i0], row0.at[pl.ds(0, N)],
                                  sem0).wait()
            process(i0, row0)

            @pl.when(2 * rp + 2 < RPW)
            def _():
                pltpu.async_copy(s_hbm.at[i1 + NW], row0.at[pl.ds(0, N)],
                                 sem0)

            pltpu.make_async_copy(s_hbm.at[i1], row1.at[pl.ds(0, N)],
                                  sem1).wait()
            process(i1, row1)
            return carry

        lax.fori_loop(0, RPW // 2, pair, 0)

    return k(ssum)


@jax.jit
def kernel(mentions):
    s = _ssum(mentions)
    vals, idxs = _topk_sc(s)
    return vals[:, :K], idxs[:, :K]


# speedup vs baseline: 1.5902x; 1.5902x over previous
import functools

import jax
import jax.numpy as jnp
from jax import lax
from jax.experimental import pallas as pl
from jax.experimental.pallas import tpu as pltpu
from jax.experimental.pallas import tpu_sc as plsc

K = 50
N = 8192
NW = 32
RPW = N // NW
PADK = 128
NEG = float("-inf")
BIG = 1 << 30
RB = 256
CAPA = 576
SEG = 256
BOOT = 128
ROWPAD = N + SEG

BT = 512


def _ssum_body(a_ref, b_ref, o_ref):
    o_ref[...] = a_ref[...] + b_ref[...].T


def _ssum(m):
    g = N // BT
    return pl.pallas_call(
        _ssum_body,
        grid=(g, g),
        in_specs=[
            pl.BlockSpec((BT, BT), lambda i, j: (i, j)),
            pl.BlockSpec((BT, BT), lambda i, j: (j, i)),
        ],
        out_specs=pl.BlockSpec((BT, BT), lambda i, j: (i, j)),
        out_shape=jax.ShapeDtypeStruct((N, N), jnp.float32),
    )(m, m)


def _iota16():
    return lax.iota(jnp.int32, 16)


def _topk_sc(ssum):
    mesh = plsc.VectorSubcoreMesh(core_axis_name="c", subcore_axis_name="s")

    @functools.partial(
        pl.kernel,
        mesh=mesh,
        out_type=[
            jax.ShapeDtypeStruct((N, PADK), jnp.float32),
            jax.ShapeDtypeStruct((N, PADK), jnp.int32),
        ],
        scratch_types=[
            pltpu.VMEM((ROWPAD,), jnp.float32),
            pltpu.VMEM((ROWPAD,), jnp.float32),
            pltpu.VMEM((CAPA,), jnp.float32),
            pltpu.VMEM((CAPA,), jnp.int32),
            pltpu.VMEM((PADK,), jnp.float32),
            pltpu.VMEM((PADK,), jnp.int32),
            pltpu.SemaphoreType.DMA,
            pltpu.SemaphoreType.DMA,
        ],
        compiler_params=pltpu.CompilerParams(needs_layout_passes=False),
    )
    def k(s_hbm, ov_hbm, oi_hbm, row0, row1, bufv, bufi, outv, outi,
          sem0, sem1):
        wid = lax.axis_index("s") * 2 + lax.axis_index("c")
        lane0 = _iota16() == 0

        def extract(ptr):
            bufv[pl.ds(ptr, 16)] = jnp.full((16,), NEG, jnp.float32)
            bufv[pl.ds(ptr + 16, 16)] = jnp.full((16,), NEG, jnp.float32)
            npair = ptr // 32 + 1

            def one(kk, _):
                def scan(g, c):
                    bva, bpa, bvb, bpb = c
                    xa = bufv[pl.ds(g * 32, 16)]
                    cia = bufi[pl.ds(g * 32, 16)]
                    xb = bufv[pl.ds(g * 32 + 16, 16)]
                    cib = bufi[pl.ds(g * 32 + 16, 16)]
                    sla = g * 32 + _iota16()
                    ma = xa > bva
                    mb = xb > bvb
                    return (jnp.where(ma, xa, bva),
                            jnp.where(ma, sla * 16384 + cia, bpa),
                            jnp.where(mb, xb, bvb),
                            jnp.where(mb, (sla + 16) * 16384 + cib, bpb))

                bva, bpa, bvb, bpb = lax.fori_loop(
                    0, npair, scan,
                    (jnp.full((16,), NEG, jnp.float32),
                     jnp.full((16,), BIG, jnp.int32),
                     jnp.full((16,), NEG, jnp.float32),
                     jnp.full((16,), BIG, jnp.int32)))
                gt = bvb > bva
                eq = bvb == bva
                bv = jnp.where(gt, bvb, bva)
                bp = jnp.where(gt, bpb,
                               jnp.where(eq, jnp.minimum(bpa, bpb), bpa))
                mx = jnp.max(bv)
                tie = bv == mx
                pmin = jnp.min(jnp.where(tie, bp, BIG))
                smin = jnp.minimum(pmin >> 14, jnp.int32(CAPA - 17))
                col = pmin & 16383
                plsc.store_scatter(bufv, [jnp.full((16,), smin, jnp.int32)],
                                   jnp.full((16,), NEG, jnp.float32), mask=lane0)
                kk16 = jnp.full((16,), kk, jnp.int32)
                plsc.store_scatter(outv, [kk16],
                                   jnp.full((16,), mx, jnp.float32), mask=lane0)
                plsc.store_scatter(outi, [kk16],
                                   jnp.full((16,), col, jnp.int32), mask=lane0)
                return mx

            return lax.fori_loop(0, K, one, jnp.float32(NEG))

        def rebuild(ptr):
            thr = extract(ptr)
            for g in range(4):
                bufv[pl.ds(g * 16, 16)] = outv[pl.ds(g * 16, 16)]
                bufi[pl.ds(g * 16, 16)] = outi[pl.ds(g * 16, 16)]
            return thr, jnp.int32(K)

        def append(x, cols, mask, ptr):
            cnt = mask.astype(jnp.int32)
            pos = ptr + plsc.cumsum(cnt) - 1
            plsc.store_scatter(bufv, [pos], x, mask=mask)
            plsc.store_scatter(bufi, [pos], cols, mask=mask)
            return ptr + jnp.sum(cnt)

        def process(i, row):
            for t in range(16):
                row[pl.ds(i + t * 16, 16)] = jnp.full((16,), NEG,
                                                      jnp.float32)
            thr = jnp.float32(NEG)
            ptr = jnp.int32(0)
            for g in range(BOOT // 16):
                x = row[pl.ds(g * 16, 16)]
                ptr = append(x, g * 16 + _iota16(), x > thr, ptr)
            thr, ptr = rebuild(ptr)

            nseg = jnp.maximum(0, (i - BOOT + SEG - 1) // SEG)

            def seg_body(s, c):
                thr, ptr = c
                base = BOOT + s * SEG
                for g in range(SEG // 16):
                    x = row[pl.ds(base + g * 16, 16)]
                    ptr = append(x, base + g * 16 + _iota16(), x > thr, ptr)
                return lax.cond(ptr >= RB, lambda: rebuild(ptr),
                                lambda: (thr, ptr))

            thr, ptr = lax.fori_loop(0, nseg, seg_body, (thr, ptr))
            extract(ptr)

            for g in range(PADK // 16):
                pv = g * 16 + _iota16()
                pm = pv >= i
                outv[pl.ds(g * 16, 16)] = jnp.where(
                    pm, jnp.full((16,), NEG, jnp.float32),
                    outv[pl.ds(g * 16, 16)])
                outi[pl.ds(g * 16, 16)] = jnp.where(
                    pm, pv, outi[pl.ds(g * 16, 16)])
            pltpu.sync_copy(outv, ov_hbm.at[i])
            pltpu.sync_copy(outi, oi_hbm.at[i])

        pltpu.async_copy(s_hbm.at[wid], row0.at[pl.ds(0, N)], sem0)

        def pair(rp, carry):
            i0 = wid + NW * 2 * rp
            i1 = i0 + NW
            pltpu.async_copy(s_hbm.at[i1], row1.at[pl.ds(0, N)], sem1)
            pltpu.make_async_copy(s_hbm.at[i0], row0.at[pl.ds(0, N)],
                                  sem0).wait()
            process(i0, row0)

            @pl.when(2 * rp + 2 < RPW)
            def _():
                pltpu.async_copy(s_hbm.at[i1 + NW], row0.at[pl.ds(0, N)],
                                 sem0)

            pltpu.make_async_copy(s_hbm.at[i1], row1.at[pl.ds(0, N)],
                                  sem1).wait()
            process(i1, row1)
            return carry

        lax.fori_loop(0, RPW // 2, pair, 0)

    return k(ssum)


@jax.jit
def kernel(mentions):
    s = _ssum(mentions)
    vals, idxs = _topk_sc(s)
    return vals[:, :K], idxs[:, :K]


# speedup vs baseline: 1.5942x; 1.0025x over previous
import functools

import jax
import jax.numpy as jnp
from jax import lax
from jax.experimental import pallas as pl
from jax.experimental.pallas import tpu as pltpu
from jax.experimental.pallas import tpu_sc as plsc

K = 50
N = 8192
NW = 32
RPW = N // NW
PADK = 128
NEG = float("-inf")
BIG = 1 << 30
RB = 256
CAPA = 576
SEG = 256
BOOT = 128
ROWPAD = N + SEG

BT = 512


def _ssum_body(a_ref, b_ref, o_ref):
    o_ref[...] = a_ref[...] + b_ref[...].T


def _ssum(m):
    g = N // BT
    return pl.pallas_call(
        _ssum_body,
        grid=(g, g),
        in_specs=[
            pl.BlockSpec((BT, BT), lambda i, j: (i, j)),
            pl.BlockSpec((BT, BT), lambda i, j: (j, i)),
        ],
        out_specs=pl.BlockSpec((BT, BT), lambda i, j: (i, j)),
        out_shape=jax.ShapeDtypeStruct((N, N), jnp.float32),
    )(m, m)


def _iota16():
    return lax.iota(jnp.int32, 16)


def _topk_sc(ssum):
    mesh = plsc.VectorSubcoreMesh(core_axis_name="c", subcore_axis_name="s")

    @functools.partial(
        pl.kernel,
        mesh=mesh,
        out_type=[
            jax.ShapeDtypeStruct((N, PADK), jnp.float32),
            jax.ShapeDtypeStruct((N, PADK), jnp.int32),
        ],
        scratch_types=[
            pltpu.VMEM((ROWPAD,), jnp.float32),
            pltpu.VMEM((ROWPAD,), jnp.float32),
            pltpu.VMEM((CAPA,), jnp.float32),
            pltpu.VMEM((CAPA,), jnp.int32),
            pltpu.VMEM((PADK,), jnp.float32),
            pltpu.VMEM((PADK,), jnp.int32),
            pltpu.SemaphoreType.DMA,
            pltpu.SemaphoreType.DMA,
        ],
        compiler_params=pltpu.CompilerParams(needs_layout_passes=False),
    )
    def k(s_hbm, ov_hbm, oi_hbm, row0, row1, bufv, bufi, outv, outi,
          sem0, sem1):
        wid = lax.axis_index("s") * 2 + lax.axis_index("c")
        lane0 = _iota16() == 0

        def extract(ptr):
            for t in range(4):
                bufv[pl.ds(ptr + t * 16, 16)] = jnp.full((16,), NEG,
                                                         jnp.float32)
            nquad = ptr // 64 + 1

            def one(kk, _):
                def scan(g, c):
                    def lane(t, bv, bp):
                        x = bufv[pl.ds(g * 64 + t * 16, 16)]
                        ci = bufi[pl.ds(g * 64 + t * 16, 16)]
                        sl = g * 64 + t * 16 + _iota16()
                        m = x > bv
                        return (jnp.where(m, x, bv),
                                jnp.where(m, sl * 16384 + ci, bp))
                    return (*lane(0, c[0], c[1]), *lane(1, c[2], c[3]),
                            *lane(2, c[4], c[5]), *lane(3, c[6], c[7]))

                init = []
                for _t in range(4):
                    init += [jnp.full((16,), NEG, jnp.float32),
                             jnp.full((16,), BIG, jnp.int32)]
                r = lax.fori_loop(0, nquad, scan, tuple(init))

                def merge(bva, bpa, bvb, bpb):
                    gt = bvb > bva
                    eq = bvb == bva
                    return (jnp.where(gt, bvb, bva),
                            jnp.where(gt, bpb,
                                      jnp.where(eq, jnp.minimum(bpa, bpb),
                                                bpa)))

                bv0, bp0 = merge(r[0], r[1], r[2], r[3])
                bv1, bp1 = merge(r[4], r[5], r[6], r[7])
                bv, bp = merge(bv0, bp0, bv1, bp1)
                mx = jnp.max(bv)
                tie = bv == mx
                pmin = jnp.min(jnp.where(tie, bp, BIG))
                smin = jnp.minimum(pmin >> 14, jnp.int32(CAPA - 17))
                col = pmin & 16383
                plsc.store_scatter(bufv, [jnp.full((16,), smin, jnp.int32)],
                                   jnp.full((16,), NEG, jnp.float32), mask=lane0)
                kk16 = jnp.full((16,), kk, jnp.int32)
                plsc.store_scatter(outv, [kk16],
                                   jnp.full((16,), mx, jnp.float32), mask=lane0)
                plsc.store_scatter(outi, [kk16],
                                   jnp.full((16,), col, jnp.int32), mask=lane0)
                return mx

            return lax.fori_loop(0, K, one, jnp.float32(NEG))

        def rebuild(ptr):
            thr = extract(ptr)
            for g in range(4):
                bufv[pl.ds(g * 16, 16)] = outv[pl.ds(g * 16, 16)]
                bufi[pl.ds(g * 16, 16)] = outi[pl.ds(g * 16, 16)]
            return thr, jnp.int32(K)

        def append(x, cols, mask, ptr):
            cnt = mask.astype(jnp.int32)
            pos = ptr + plsc.cumsum(cnt) - 1
            plsc.store_scatter(bufv, [pos], x, mask=mask)
            plsc.store_scatter(bufi, [pos], cols, mask=mask)
            return ptr + jnp.sum(cnt)

        def process(i, row):
            for t in range(16):
                row[pl.ds(i + t * 16, 16)] = jnp.full((16,), NEG,
                                                      jnp.float32)
            thr = jnp.float32(NEG)
            ptr = jnp.int32(0)
            for g in range(BOOT // 16):
                x = row[pl.ds(g * 16, 16)]
                ptr = append(x, g * 16 + _iota16(), x > thr, ptr)
            thr, ptr = rebuild(ptr)

            nseg = jnp.maximum(0, (i - BOOT + SEG - 1) // SEG)

            def seg_body(s, c):
                thr, ptr = c
                base = BOOT + s * SEG
                for g in range(SEG // 16):
                    x = row[pl.ds(base + g * 16, 16)]
                    ptr = append(x, base + g * 16 + _iota16(), x > thr, ptr)
                return lax.cond(ptr >= RB, lambda: rebuild(ptr),
                                lambda: (thr, ptr))

            thr, ptr = lax.fori_loop(0, nseg, seg_body, (thr, ptr))
            extract(ptr)

            for g in range(PADK // 16):
                pv = g * 16 + _iota16()
                pm = pv >= i
                outv[pl.ds(g * 16, 16)] = jnp.where(
                    pm, jnp.full((16,), NEG, jnp.float32),
                    outv[pl.ds(g * 16, 16)])
                outi[pl.ds(g * 16, 16)] = jnp.where(
                    pm, pv, outi[pl.ds(g * 16, 16)])
            pltpu.sync_copy(outv, ov_hbm.at[i])
            pltpu.sync_copy(outi, oi_hbm.at[i])

        pltpu.async_copy(s_hbm.at[wid], row0.at[pl.ds(0, N)], sem0)

        def pair(rp, carry):
            i0 = wid + NW * 2 * rp
            i1 = i0 + NW
            pltpu.async_copy(s_hbm.at[i1], row1.at[pl.ds(0, N)], sem1)
            pltpu.make_async_copy(s_hbm.at[i0], row0.at[pl.ds(0, N)],
                                  sem0).wait()
            process(i0, row0)

            @pl.when(2 * rp + 2 < RPW)
            def _():
                pltpu.async_copy(s_hbm.at[i1 + NW], row0.at[pl.ds(0, N)],
                                 sem0)

            pltpu.make_async_copy(s_hbm.at[i1], row1.at[pl.ds(0, N)],
                                  sem1).wait()
            process(i1, row1)
            return carry

        lax.fori_loop(0, RPW // 2, pair, 0)

    return k(ssum)


@jax.jit
def kernel(mentions):
    s = _ssum(mentions)
    vals, idxs = _topk_sc(s)
    return vals[:, :K], idxs[:, :K]


# speedup vs baseline: 1.8842x; 1.1819x over previous
import functools

import jax
import jax.numpy as jnp
from jax import lax
from jax.experimental import pallas as pl
from jax.experimental.pallas import tpu as pltpu
from jax.experimental.pallas import tpu_sc as plsc

K = 50
N = 8192
NW = 32
RPW = N // NW
PADK = 128
NEG = float("-inf")
BIG = 1 << 30
RB = 256
CAPA = 576
SEG = 256
BOOT = 128
ROWPAD = N + SEG

BT = 512


def _ssum_body(a_ref, b_ref, o_ref):
    o_ref[...] = a_ref[...] + b_ref[...].T


def _ssum(m):
    g = N // BT
    return pl.pallas_call(
        _ssum_body,
        grid=(g, g),
        in_specs=[
            pl.BlockSpec((BT, BT), lambda i, j: (i, j)),
            pl.BlockSpec((BT, BT), lambda i, j: (j, i)),
        ],
        out_specs=pl.BlockSpec((BT, BT), lambda i, j: (i, j)),
        out_shape=jax.ShapeDtypeStruct((N, N), jnp.float32),
    )(m, m)


def _iota16():
    return lax.iota(jnp.int32, 16)


def _topk_sc(ssum):
    mesh = plsc.VectorSubcoreMesh(core_axis_name="c", subcore_axis_name="s")

    @functools.partial(
        pl.kernel,
        mesh=mesh,
        out_type=[
            jax.ShapeDtypeStruct((N, PADK), jnp.float32),
            jax.ShapeDtypeStruct((N, PADK), jnp.int32),
        ],
        scratch_types=[
            pltpu.VMEM((ROWPAD,), jnp.float32),
            pltpu.VMEM((ROWPAD,), jnp.float32),
            pltpu.VMEM((CAPA,), jnp.float32),
            pltpu.VMEM((CAPA,), jnp.int32),
            pltpu.VMEM((PADK,), jnp.float32),
            pltpu.VMEM((PADK,), jnp.int32),
            pltpu.SemaphoreType.DMA,
            pltpu.SemaphoreType.DMA,
        ],
        compiler_params=pltpu.CompilerParams(needs_layout_passes=False),
    )
    def k(s_hbm, ov_hbm, oi_hbm, row0, row1, bufv, bufi, outv, outi,
          sem0, sem1):
        wid = lax.axis_index("s") * 2 + lax.axis_index("c")
        lane0 = _iota16() == 0

        def extract(ptr):
            for t in range(4):
                bufv[pl.ds(ptr + t * 16, 16)] = jnp.full((16,), NEG,
                                                         jnp.float32)
            nquad = ptr // 64 + 1

            def one(kk, _):
                def scan(g, c):
                    def lane(t, bv, bp):
                        x = bufv[pl.ds(g * 64 + t * 16, 16)]
                        ci = bufi[pl.ds(g * 64 + t * 16, 16)]
                        sl = g * 64 + t * 16 + _iota16()
                        m = x > bv
                        return (jnp.where(m, x, bv),
                                jnp.where(m, sl * 16384 + ci, bp))
                    return (*lane(0, c[0], c[1]), *lane(1, c[2], c[3]),
                            *lane(2, c[4], c[5]), *lane(3, c[6], c[7]))

                init = []
                for _t in range(4):
                    init += [jnp.full((16,), NEG, jnp.float32),
                             jnp.full((16,), BIG, jnp.int32)]
                r = lax.fori_loop(0, nquad, scan, tuple(init))

                def merge(bva, bpa, bvb, bpb):
                    gt = bvb > bva
                    eq = bvb == bva
                    return (jnp.where(gt, bvb, bva),
                            jnp.where(gt, bpb,
                                      jnp.where(eq, jnp.minimum(bpa, bpb),
                                                bpa)))

                bv0, bp0 = merge(r[0], r[1], r[2], r[3])
                bv1, bp1 = merge(r[4], r[5], r[6], r[7])
                bv, bp = merge(bv0, bp0, bv1, bp1)
                mx = jnp.max(bv)
                tie = bv == mx
                pmin = jnp.min(jnp.where(tie, bp, BIG))
                smin = jnp.minimum(pmin >> 14, jnp.int32(CAPA - 17))
                col = pmin & 16383
                plsc.store_scatter(bufv, [jnp.full((16,), smin, jnp.int32)],
                                   jnp.full((16,), NEG, jnp.float32), mask=lane0)
                kk16 = jnp.full((16,), kk, jnp.int32)
                plsc.store_scatter(outv, [kk16],
                                   jnp.full((16,), mx, jnp.float32), mask=lane0)
                plsc.store_scatter(outi, [kk16],
                                   jnp.full((16,), col, jnp.int32), mask=lane0)
                return mx

            return lax.fori_loop(0, K, one, jnp.float32(NEG))

        def rebuild(ptr):
            thr = extract(ptr)
            for g in range(4):
                bufv[pl.ds(g * 16, 16)] = outv[pl.ds(g * 16, 16)]
                bufi[pl.ds(g * 16, 16)] = outi[pl.ds(g * 16, 16)]
            return jnp.full((16,), thr, jnp.float32), jnp.int32(K)

        def append(x, cols, mask, ptr):
            cnt = mask.astype(jnp.int32)
            pos = ptr + plsc.cumsum(cnt) - 1
            plsc.store_scatter(bufv, [pos], x, mask=mask)
            plsc.store_scatter(bufi, [pos], cols, mask=mask)
            return ptr + jnp.sum(cnt)

        def process(i, row):
            for t in range(16):
                row[pl.ds(i + t * 16, 16)] = jnp.full((16,), NEG,
                                                      jnp.float32)
            def unkey16(bits):
                b16 = jnp.full((16,), bits, jnp.int32)
                return lax.bitcast_convert_type(
                    jnp.where(b16 >= 0, b16, b16 ^ 0x7FFFFFFF), jnp.float32)

            def bis(_, c):
                lo, hi = c
                mid = (lo >> 1) + (hi >> 1)
                tf = unkey16(mid)
                acc = jnp.zeros((16,), jnp.int32)
                for g in range(BOOT // 16):
                    acc = acc + jnp.where(row[pl.ds(g * 16, 16)] > tf, 1, 0)
                ok = jnp.sum(acc) >= K
                return (jnp.where(ok, mid, lo), jnp.where(ok, hi, mid))

            lo, _hi = lax.fori_loop(
                0, 18, bis,
                (jnp.int32(-2139095041), jnp.int32(2139095041)))
            thr = unkey16(lo)
            ptr = jnp.int32(0)
            for g in range(BOOT // 16):
                x = row[pl.ds(g * 16, 16)]
                ptr = append(x, g * 16 + _iota16(), x > thr, ptr)

            nseg = jnp.maximum(0, (i - BOOT + SEG - 1) // SEG)

            def seg_body(s, c):
                thr, ptr = c
                base = BOOT + s * SEG
                for g in range(SEG // 16):
                    x = row[pl.ds(base + g * 16, 16)]
                    ptr = append(x, base + g * 16 + _iota16(), x > thr, ptr)
                return lax.cond(ptr >= RB, lambda: rebuild(ptr),
                                lambda: (thr, ptr))

            thr, ptr = lax.fori_loop(0, nseg, seg_body, (thr, ptr))
            extract(ptr)

            for g in range(PADK // 16):
                pv = g * 16 + _iota16()
                pm = pv >= i
                outv[pl.ds(g * 16, 16)] = jnp.where(
                    pm, jnp.full((16,), NEG, jnp.float32),
                    outv[pl.ds(g * 16, 16)])
                outi[pl.ds(g * 16, 16)] = jnp.where(
                    pm, pv, outi[pl.ds(g * 16, 16)])
            pltpu.sync_copy(outv, ov_hbm.at[i])
            pltpu.sync_copy(outi, oi_hbm.at[i])

        pltpu.async_copy(s_hbm.at[wid], row0.at[pl.ds(0, N)], sem0)

        def pair(rp, carry):
            i0 = wid + NW * 2 * rp
            i1 = i0 + NW
            pltpu.async_copy(s_hbm.at[i1], row1.at[pl.ds(0, N)], sem1)
            pltpu.make_async_copy(s_hbm.at[i0], row0.at[pl.ds(0, N)],
                                  sem0).wait()
            process(i0, row0)

            @pl.when(2 * rp + 2 < RPW)
            def _():
                pltpu.async_copy(s_hbm.at[i1 + NW], row0.at[pl.ds(0, N)],
                                 sem0)

            pltpu.make_async_copy(s_hbm.at[i1], row1.at[pl.ds(0, N)],
                                  sem1).wait()
            process(i1, row1)
            return carry

        lax.fori_loop(0, RPW // 2, pair, 0)

    return k(ssum)


@jax.jit
def kernel(mentions):
    s = _ssum(mentions)
    vals, idxs = _topk_sc(s)
    return vals[:, :K], idxs[:, :K]
